# Initial kernel scaffold; baseline (speedup 1.0000x reference)
import jax
import jax.numpy as jnp
from jax.experimental import pallas as pl


def kernel(x, table, W):
    raise NotImplementedError("write your pallas kernel here")


# speedup vs baseline: 7.9266x; 7.9266x over previous
import functools

import jax
import jax.numpy as jnp
from jax import lax
from jax.experimental import pallas as pl
from jax.experimental.pallas import tpu as pltpu
from jax.experimental.pallas import tpu_sc as plsc

OUT_DIM = 32
IN_DIM = 128

_PROJ_BLK = 8000


def _proj_body(t_ref, w_ref, o_ref):
    o_ref[...] = lax.dot_general(
        t_ref[...], w_ref[...],
        dimension_numbers=(((1,), (1,)), ((), ())),
        preferred_element_type=jnp.float32)


def _project(table, W):
    V, D = table.shape
    O = W.shape[0]
    blk = _PROJ_BLK if V % _PROJ_BLK == 0 else V
    return pl.pallas_call(
        _proj_body,
        grid=(V // blk,),
        in_specs=[
            pl.BlockSpec((blk, D), lambda i: (i, 0)),
            pl.BlockSpec((O, D), lambda i: (0, 0)),
        ],
        out_specs=pl.BlockSpec((blk, O), lambda i: (i, 0)),
        out_shape=jax.ShapeDtypeStruct((V, O), jnp.float32),
    )(table, W)


_NC = 2
_NS = 16
_NW = _NC * _NS
_C = 128
_K = 8


def _make_gather(N, O):
    assert N % (_NW * _C * _K) == 0
    rows_pw = N // _NW
    n_chunks = rows_pw // _C
    n_groups = n_chunks // _K
    grp_rows = _K * _C

    mesh = plsc.VectorSubcoreMesh(
        core_axis_name="c", subcore_axis_name="s",
        num_cores=_NC, num_subcores=_NS)

    @functools.partial(
        pl.kernel,
        out_type=jax.ShapeDtypeStruct((N, O), jnp.float32),
        mesh=mesh,
        scratch_types=[
            pltpu.VMEM((n_chunks, _C), jnp.int32),
            pltpu.VMEM((2, grp_rows, O), jnp.float32),
            pltpu.SemaphoreType.DMA,
            pltpu.SemaphoreType.DMA,
        ],
        compiler_params=pltpu.CompilerParams(use_tc_tiling_on_sc=False),
    )
    def gather_k(idx_hbm, small_hbm, out_hbm, idx_v, rows_v, gsem, osem):
        wid = lax.axis_index("s") * _NC + lax.axis_index("c")
        pltpu.sync_copy(idx_hbm.at[pl.ds(wid * n_chunks, n_chunks)], idx_v)
        base = wid * rows_pw

        def fire_gathers(g):
            buf = rows_v.at[lax.rem(g, 2)]
            for j in range(_K):
                pltpu.async_copy(
                    small_hbm.at[idx_v.at[g * _K + j]],
                    buf.at[pl.ds(j * _C, _C)],
                    gsem)

        def drain_gathers(g):
            pltpu.make_async_copy(
                small_hbm.at[pl.ds(0, grp_rows)],
                rows_v.at[lax.rem(g, 2)],
                gsem).wait()

        def start_out_copy(g):
            pltpu.async_copy(
                rows_v.at[lax.rem(g, 2)],
                out_hbm.at[pl.ds(base + g * grp_rows, grp_rows)],
                osem)

        def drain_out_copy(g):
            pltpu.make_async_copy(
                rows_v.at[lax.rem(g, 2)],
                out_hbm.at[pl.ds(base + g * grp_rows, grp_rows)],
                osem).wait()

        def body(g, carry):
            @pl.when(g >= 2)
            def _():
                drain_out_copy(g - 2)
            fire_gathers(g)
            @pl.when(g >= 1)
            def _():
                drain_gathers(g - 1)
                start_out_copy(g - 1)
            return carry

        lax.fori_loop(0, n_groups, body, 0)
        drain_gathers(n_groups - 1)
        start_out_copy(n_groups - 1)
        drain_out_copy(n_groups - 2)
        drain_out_copy(n_groups - 1)

    return gather_k


def kernel(x, table, W):
    B, L = x.shape
    O = W.shape[0]
    N = B * L
    small = _project(table, W)
    idx2d = x.astype(jnp.int32).reshape(N // _C, _C)
    out_flat = _make_gather(N, O)(idx2d, small)
    return out_flat.reshape(B, L, O)


# speedup vs baseline: 12.1432x; 1.5319x over previous
import functools

import jax
import jax.numpy as jnp
from jax import lax
from jax.experimental import pallas as pl
from jax.experimental.pallas import tpu as pltpu
from jax.experimental.pallas import tpu_sc as plsc

OUT_DIM = 32
IN_DIM = 128

_PROJ_BLK = 8000


def _proj_body(t_ref, w_ref, o_ref):
    o_ref[...] = lax.dot_general(
        t_ref[...], w_ref[...],
        dimension_numbers=(((1,), (1,)), ((), ())),
        preferred_element_type=jnp.float32)


def _project(table, W):
    V, D = table.shape
    O = W.shape[0]
    blk = _PROJ_BLK if V % _PROJ_BLK == 0 else V
    return pl.pallas_call(
        _proj_body,
        grid=(V // blk,),
        in_specs=[
            pl.BlockSpec((blk, D), lambda i: (i, 0)),
            pl.BlockSpec((O, D), lambda i: (0, 0)),
        ],
        out_specs=pl.BlockSpec((blk, O), lambda i: (i, 0)),
        out_shape=jax.ShapeDtypeStruct((V, O), jnp.float32),
    )(table, W)


_NC = 2
_NS = 16
_NW = _NC * _NS
_K = 16


def _make_gather(B, L, O):
    assert B % (_NW * _K) == 0
    xrows_pw = B // _NW
    n_groups = xrows_pw // _K

    mesh = plsc.VectorSubcoreMesh(
        core_axis_name="c", subcore_axis_name="s",
        num_cores=_NC, num_subcores=_NS)

    @functools.partial(
        pl.kernel,
        out_type=jax.ShapeDtypeStruct((B, L, O), jnp.float32),
        mesh=mesh,
        scratch_types=[
            pltpu.VMEM((xrows_pw, L), jnp.int32),
            pltpu.VMEM((2, _K, L, O), jnp.float32),
            pltpu.SemaphoreType.DMA,
            pltpu.SemaphoreType.DMA,
        ],
        compiler_params=pltpu.CompilerParams(use_tc_tiling_on_sc=False),
    )
    def gather_k(x_hbm, small_hbm, out_hbm, idx_v, rows_v, gsem, osem):
        wid = lax.axis_index("s") * _NC + lax.axis_index("c")
        base = wid * xrows_pw
        pltpu.sync_copy(x_hbm.at[pl.ds(base, xrows_pw)], idx_v)

        def fire_gathers(g):
            buf = rows_v.at[lax.rem(g, 2)]
            for j in range(_K):
                pltpu.async_copy(
                    small_hbm.at[idx_v.at[g * _K + j]],
                    buf.at[j],
                    gsem)

        def drain_gathers(g):
            pltpu.make_async_copy(
                out_hbm.at[pl.ds(0, _K)],
                rows_v.at[lax.rem(g, 2)],
                gsem).wait()

        def start_out_copy(g):
            pltpu.async_copy(
                rows_v.at[lax.rem(g, 2)],
                out_hbm.at[pl.ds(base + g * _K, _K)],
                osem)

        def drain_out_copy(g):
            pltpu.make_async_copy(
                rows_v.at[lax.rem(g, 2)],
                out_hbm.at[pl.ds(base + g * _K, _K)],
                osem).wait()

        def body(g, carry):
            @pl.when(g >= 2)
            def _():
                drain_out_copy(g - 2)
            fire_gathers(g)
            @pl.when(g >= 1)
            def _():
                drain_gathers(g - 1)
                start_out_copy(g - 1)
            return carry

        lax.fori_loop(0, n_groups, body, 0)
        drain_gathers(n_groups - 1)
        start_out_copy(n_groups - 1)
        drain_out_copy(n_groups - 2)
        drain_out_copy(n_groups - 1)

    return gather_k


def kernel(x, table, W):
    B, L = x.shape
    O = W.shape[0]
    small = _project(table, W)
    return _make_gather(B, L, O)(x.astype(jnp.int32), small)


# speedup vs baseline: 18.2850x; 1.5058x over previous
import functools

import jax
import jax.numpy as jnp
from jax import lax
from jax.experimental import pallas as pl
from jax.experimental.pallas import tpu as pltpu
from jax.experimental.pallas import tpu_sc as plsc

OUT_DIM = 32
IN_DIM = 128

_PROJ_BLK = 8000


def _proj_body(t0, t1, t2, t3, w_ref, o_ref):
    dn = (((1,), (1,)), ((), ()))
    o_ref[...] = jnp.concatenate(
        [lax.dot_general(t[...], w_ref[...], dimension_numbers=dn,
                         preferred_element_type=jnp.float32)
         for t in (t0, t1, t2, t3)],
        axis=1)


def _project(table, W):
    V, D = table.shape
    O = W.shape[0]
    pack = 128 // O
    Q = V // pack
    blk = 2000
    nblk = Q // blk

    def tspec(q):
        return pl.BlockSpec((blk, D), lambda i, q=q: (i + q * nblk, 0))

    packed = pl.pallas_call(
        _proj_body,
        grid=(nblk,),
        in_specs=[tspec(0), tspec(1), tspec(2), tspec(3),
                  pl.BlockSpec((O, D), lambda i: (0, 0))],
        out_specs=pl.BlockSpec((blk, O * pack), lambda i: (i, 0)),
        out_shape=jax.ShapeDtypeStruct((Q, O * pack), jnp.float32),
    )(table, table, table, table, W)
    return packed.reshape(V, O)


_NC = 2
_NS = 16
_NW = _NC * _NS
_K = 16


def _make_gather(B, L, O):
    assert B % (_NW * _K) == 0
    xrows_pw = B // _NW
    n_groups = xrows_pw // _K

    mesh = plsc.VectorSubcoreMesh(
        core_axis_name="c", subcore_axis_name="s",
        num_cores=_NC, num_subcores=_NS)

    @functools.partial(
        pl.kernel,
        out_type=jax.ShapeDtypeStruct((B, L, O), jnp.float32),
        mesh=mesh,
        scratch_types=[
            pltpu.VMEM((xrows_pw, L), jnp.int32),
            pltpu.VMEM((2, _K, L, O), jnp.float32),
            pltpu.SemaphoreType.DMA,
            pltpu.SemaphoreType.DMA,
        ],
        compiler_params=pltpu.CompilerParams(use_tc_tiling_on_sc=False),
    )
    def gather_k(x_hbm, small_hbm, out_hbm, idx_v, rows_v, gsem, osem):
        wid = lax.axis_index("s") * _NC + lax.axis_index("c")
        base = wid * xrows_pw
        pltpu.sync_copy(x_hbm.at[pl.ds(base, xrows_pw)], idx_v)

        def fire_gathers(g):
            buf = rows_v.at[lax.rem(g, 2)]
            for j in range(_K):
                pltpu.async_copy(
                    small_hbm.at[idx_v.at[g * _K + j]],
                    buf.at[j],
                    gsem)

        def drain_gathers(g):
            pltpu.make_async_copy(
                out_hbm.at[pl.ds(0, _K)],
                rows_v.at[lax.rem(g, 2)],
                gsem).wait()

        def start_out_copy(g):
            pltpu.async_copy(
                rows_v.at[lax.rem(g, 2)],
                out_hbm.at[pl.ds(base + g * _K, _K)],
                osem)

        def drain_out_copy(g):
            pltpu.make_async_copy(
                rows_v.at[lax.rem(g, 2)],
                out_hbm.at[pl.ds(base + g * _K, _K)],
                osem).wait()

        def body(g, carry):
            @pl.when(g >= 2)
            def _():
                drain_out_copy(g - 2)
            fire_gathers(g)
            @pl.when(g >= 1)
            def _():
                drain_gathers(g - 1)
                start_out_copy(g - 1)
            return carry

        lax.fori_loop(0, n_groups, body, 0)
        drain_gathers(n_groups - 1)
        start_out_copy(n_groups - 1)
        drain_out_copy(n_groups - 2)
        drain_out_copy(n_groups - 1)

    return gather_k


def kernel(x, table, W):
    B, L = x.shape
    V = table.shape[0]
    O = W.shape[0]
    small = _project(table, W)
    Q = V // (128 // O)
    xi = x.astype(jnp.int32)
    q = xi // Q
    xr = (xi - q * Q) * (128 // O) + q
    return _make_gather(B, L, O)(xr, small)
